# baseline (speedup 1.0000x reference)
import functools

import jax
import jax.numpy as jnp
import numpy as np
from jax import lax
from jax.experimental import pallas as pl
from jax.experimental.pallas import tpu as pltpu
from jax.experimental.pallas import tpu_sc as plsc

_VOCAB = 100000
_D = 768
_B, _S = 4, 4096
_SCALE = float(np.sqrt(np.float32(_D)))

_NC = 2
_NS = 16
_NW = _NC * _NS

_POS_PER_W = _S // _NW
_CH = 32
_NCHUNK = _POS_PER_W // _CH
_NIT = _NCHUNK * _B
_CPV = _D // 16


def _make_pe(seq_len, d):
    pos = np.arange(seq_len)[:, None].astype(np.float32)
    i = np.arange(0, d, 2).astype(np.float32)
    angle = pos / np.power(10000.0, i / np.float32(d))
    pe = np.zeros((seq_len, d), dtype=np.float32)
    pe[:, 0::2] = np.sin(angle)
    pe[:, 1::2] = np.cos(angle)
    return pe


_PE_HOST = _make_pe(_S, _D)


@functools.partial(
    pl.kernel,
    out_type=jax.ShapeDtypeStruct((_B * _S, _D), jnp.float32),
    mesh=plsc.VectorSubcoreMesh(core_axis_name="c", subcore_axis_name="s"),
    scratch_types=[
        pltpu.VMEM((_B, _POS_PER_W), jnp.int32),
        pltpu.VMEM((_CH, _D), jnp.float32),
        pltpu.VMEM((_CH, _D), jnp.float32),
        pltpu.VMEM((_CH, _D), jnp.float32),
        pltpu.VMEM((_CH, _D), jnp.float32),
        pltpu.VMEM((_CH, _D), jnp.float32),
        pltpu.SemaphoreType.DMA,
        pltpu.SemaphoreType.DMA,
        pltpu.SemaphoreType.DMA,
        pltpu.SemaphoreType.DMA,
        pltpu.SemaphoreType.DMA,
        pltpu.SemaphoreType.DMA,
        pltpu.SemaphoreType.DMA,
        pltpu.SemaphoreType.DMA,
        pltpu.SemaphoreType.DMA,
    ],
)
def _emb_kernel(table_hbm, inp_hbm, pe_hbm, out_hbm, idx_all, rows0, rows1,
                acc0, acc1, acc2, isem, g0, g1, f0, f1, f2, s0, s1, s2):
    wid = lax.axis_index("s") * _NC + lax.axis_index("c")
    p_base = wid * _POS_PER_W
    rows = [rows0, rows1]
    acc = [acc0, acc1, acc2]
    gsem = [g0, g1]
    fsem = [f0, f1, f2]
    ssem = [s0, s1, s2]

    def idx_copy(b):
        return pltpu.make_async_copy(
            inp_hbm.at[pl.ds(b * _S + p_base, _POS_PER_W)],
            idx_all.at[b], isem)

    def fill_copy(i):
        pc = i // _B
        return pltpu.make_async_copy(
            pe_hbm.at[pl.ds(p_base + pc * _CH, _CH)],
            acc[i % 3], fsem[i % 3])

    def gather_copy(i):
        pc, b = divmod(i, _B)
        return pltpu.make_async_copy(
            table_hbm.at[idx_all.at[b, pl.ds(pc * _CH, _CH)]],
            rows[i % 2], gsem[i % 2])

    def store_copy(i):
        pc, b = divmod(i, _B)
        return pltpu.make_async_copy(
            acc[i % 3],
            out_hbm.at[pl.ds(b * _S + p_base + pc * _CH, _CH)],
            ssem[i % 3])

    for b in range(_B):
        idx_copy(b).start()
    fill_copy(0).start()
    fill_copy(1).start()
    for b in range(_B):
        idx_copy(b).wait()
    gather_copy(0).start()

    for i in range(_NIT):
        gslot = i % 2
        aslot = i % 3
        gather_copy(i).wait()
        if i + 1 < _NIT:
            gather_copy(i + 1).start()
        fill_copy(i).wait()

        @plsc.parallel_loop(0, _CH, unroll=2)
        def body(r):
            for c in range(_CPV):
                sl = pl.ds(c * 16, 16)
                plsc.addupdate(acc[aslot].at[r, sl],
                               rows[gslot][r, sl] * _SCALE)

        store_copy(i).start()
        if i + 2 < _NIT:
            if i >= 1:
                store_copy(i - 1).wait()
            fill_copy(i + 2).start()

    for i in range(_NIT - 3, _NIT):
        store_copy(i).wait()


def kernel(inp, table, is_training):
    del is_training
    pe = jnp.asarray(_PE_HOST)
    out = _emb_kernel(table, inp.reshape(_B * _S), pe)
    return out.reshape(_B, _S, _D)


# speedup vs baseline: 1.0983x; 1.0983x over previous
import functools

import jax
import jax.numpy as jnp
import numpy as np
from jax import lax
from jax.experimental import pallas as pl
from jax.experimental.pallas import tpu as pltpu
from jax.experimental.pallas import tpu_sc as plsc

_VOCAB = 100000
_D = 768
_B, _S = 4, 4096
_SCALE = float(np.sqrt(np.float32(_D)))

_NC = 2
_NS = 16
_NW = _NC * _NS

_POS_PER_W = _S // _NW
_CH = 32
_NCHUNK = _POS_PER_W // _CH
_NIT = _NCHUNK * _B
_CPV = _D // 16


def _make_pe(seq_len, d):
    pos = np.arange(seq_len)[:, None].astype(np.float32)
    i = np.arange(0, d, 2).astype(np.float32)
    angle = pos / np.power(10000.0, i / np.float32(d))
    pe = np.zeros((seq_len, d), dtype=np.float32)
    pe[:, 0::2] = np.sin(angle)
    pe[:, 1::2] = np.cos(angle)
    return pe


_PE_HOST = _make_pe(_S, _D)


@functools.partial(
    pl.kernel,
    out_type=jax.ShapeDtypeStruct((_B * _S, _D), jnp.float32),
    mesh=plsc.VectorSubcoreMesh(core_axis_name="c", subcore_axis_name="s"),
    scratch_types=[
        pltpu.VMEM((_B, _POS_PER_W), jnp.int32),
        pltpu.VMEM((_CH, _D), jnp.float32),
        pltpu.VMEM((_CH, _D), jnp.float32),
        pltpu.VMEM((_CH, _D), jnp.float32),
        pltpu.VMEM((_CH, _D), jnp.float32),
        pltpu.VMEM((_CH, _D), jnp.float32),
        pltpu.SemaphoreType.DMA,
        pltpu.SemaphoreType.DMA,
        pltpu.SemaphoreType.DMA,
        pltpu.SemaphoreType.DMA,
        pltpu.SemaphoreType.DMA,
        pltpu.SemaphoreType.DMA,
        pltpu.SemaphoreType.DMA,
        pltpu.SemaphoreType.DMA,
        pltpu.SemaphoreType.DMA,
    ],
)
def _emb_kernel(table_hbm, inp_hbm, pe_hbm, out_hbm, idx_all,
                rows0, rows1, rows2, pe0, pe1,
                isem, g0, g1, g2, s0, s1, s2, f0, f1):
    wid = lax.axis_index("s") * _NC + lax.axis_index("c")
    p_base = wid * _POS_PER_W
    rows = [rows0, rows1, rows2]
    pe = [pe0, pe1]
    gsem = [g0, g1, g2]
    ssem = [s0, s1, s2]
    fsem = [f0, f1]

    def idx_copy(b):
        return pltpu.make_async_copy(
            inp_hbm.at[pl.ds(b * _S + p_base, _POS_PER_W)],
            idx_all.at[b], isem)

    def fill_copy(pc):
        return pltpu.make_async_copy(
            pe_hbm.at[pl.ds(p_base + pc * _CH, _CH)],
            pe[pc % 2], fsem[pc % 2])

    def gather_copy(i):
        pc, b = divmod(i, _B)
        return pltpu.make_async_copy(
            table_hbm.at[idx_all.at[b, pl.ds(pc * _CH, _CH)]],
            rows[i % 3], gsem[i % 3])

    def store_copy(i):
        pc, b = divmod(i, _B)
        return pltpu.make_async_copy(
            rows[i % 3],
            out_hbm.at[pl.ds(b * _S + p_base + pc * _CH, _CH)],
            ssem[i % 3])

    for b in range(_B):
        idx_copy(b).start()
    fill_copy(0).start()
    for b in range(_B):
        idx_copy(b).wait()
    gather_copy(0).start()
    gather_copy(1).start()

    for i in range(_NIT):
        pc, b = divmod(i, _B)
        slot = i % 3
        gather_copy(i).wait()
        if b == 0:
            fill_copy(pc).wait()
            if pc + 1 < _NCHUNK:
                fill_copy(pc + 1).start()
        if i + 2 < _NIT:
            if i >= 1:
                store_copy(i - 1).wait()
            gather_copy(i + 2).start()

        @plsc.parallel_loop(0, _CH, unroll=2)
        def body(r):
            for c in range(_CPV):
                sl = pl.ds(c * 16, 16)
                rows[slot][r, sl] = (rows[slot][r, sl] * _SCALE
                                     + pe[pc % 2][r, sl])

        store_copy(i).start()

    for i in range(_NIT - 3, _NIT):
        store_copy(i).wait()


def kernel(inp, table, is_training):
    del is_training
    pe = jnp.asarray(_PE_HOST)
    out = _emb_kernel(table, inp.reshape(_B * _S), pe)
    return out.reshape(_B, _S, _D)


# speedup vs baseline: 1.1601x; 1.0563x over previous
import functools

import jax
import jax.numpy as jnp
import numpy as np
from jax import lax
from jax.experimental import pallas as pl
from jax.experimental.pallas import tpu as pltpu
from jax.experimental.pallas import tpu_sc as plsc

_VOCAB = 100000
_D = 768
_B, _S = 4, 4096
_SCALE = float(np.sqrt(np.float32(_D)))

_NC = 2
_NS = 16
_NW = _NC * _NS

_POS_PER_W = _S // _NW
_CH = 16
_NCHUNK = _POS_PER_W // _CH
_CPV = _D // 16


def _make_pe(seq_len, d):
    pos = np.arange(seq_len)[:, None].astype(np.float32)
    i = np.arange(0, d, 2).astype(np.float32)
    angle = pos / np.power(10000.0, i / np.float32(d))
    pe = np.zeros((seq_len, d), dtype=np.float32)
    pe[:, 0::2] = np.sin(angle)
    pe[:, 1::2] = np.cos(angle)
    return pe


_PE_HOST = _make_pe(_S, _D)

_ROW_SLOT = pltpu.VMEM((_CH, _D), jnp.float32)


@functools.partial(
    pl.kernel,
    out_type=jax.ShapeDtypeStruct((_B * _S, _D), jnp.float32),
    mesh=plsc.VectorSubcoreMesh(core_axis_name="c", subcore_axis_name="s"),
    scratch_types=[
        pltpu.VMEM((_B, _POS_PER_W), jnp.int32),
        _ROW_SLOT, _ROW_SLOT, _ROW_SLOT, _ROW_SLOT,
        _ROW_SLOT, _ROW_SLOT, _ROW_SLOT, _ROW_SLOT,
        _ROW_SLOT,
        _ROW_SLOT,
        pltpu.SemaphoreType.DMA,
        pltpu.SemaphoreType.DMA,
        pltpu.SemaphoreType.DMA,
        pltpu.SemaphoreType.DMA,
        pltpu.SemaphoreType.DMA,
        pltpu.SemaphoreType.DMA,
        pltpu.SemaphoreType.DMA,
    ],
)
def _emb_kernel(table_hbm, inp_hbm, pe_hbm, out_hbm, idx_all,
                r00, r01, r02, r03, r10, r11, r12, r13, pe0, pe1,
                isem, g0, g1, s0, s1, f0, f1):
    wid = lax.axis_index("s") * _NC + lax.axis_index("c")
    p_base = wid * _POS_PER_W
    rows = [[r00, r01, r02, r03], [r10, r11, r12, r13]]
    pe = [pe0, pe1]
    gsem = [g0, g1]
    ssem = [s0, s1]
    fsem = [f0, f1]

    def idx_copy(b):
        return pltpu.make_async_copy(
            inp_hbm.at[pl.ds(b * _S + p_base, _POS_PER_W)],
            idx_all.at[b], isem)

    def fill_copy(pc):
        return pltpu.make_async_copy(
            pe_hbm.at[pl.ds(p_base + pc * _CH, _CH)],
            pe[pc % 2], fsem[pc % 2])

    def gather_copy(pc, b):
        return pltpu.make_async_copy(
            table_hbm.at[idx_all.at[b, pl.ds(pc * _CH, _CH)]],
            rows[pc % 2][b], gsem[pc % 2])

    def store_copy(pc, b):
        return pltpu.make_async_copy(
            rows[pc % 2][b],
            out_hbm.at[pl.ds(b * _S + p_base + pc * _CH, _CH)],
            ssem[pc % 2])

    for b in range(_B):
        idx_copy(b).start()
    fill_copy(0).start()
    fill_copy(1).start()
    for b in range(_B):
        idx_copy(b).wait()
    for b in range(_B):
        gather_copy(0, b).start()

    for pc in range(_NCHUNK):
        grp = pc % 2
        if pc >= 1:
            for b in range(_B):
                store_copy(pc - 1, b).wait()
        if pc + 1 < _NCHUNK:
            for b in range(_B):
                gather_copy(pc + 1, b).start()
        for b in range(_B):
            gather_copy(pc, b).wait()
        fill_copy(pc).wait()

        @plsc.parallel_loop(0, _CH, unroll=2)
        def body(r):
            for c in range(_CPV):
                sl = pl.ds(c * 16, 16)
                p = pe[grp][r, sl]
                for b in range(_B):
                    rows[grp][b][r, sl] = rows[grp][b][r, sl] * _SCALE + p

        for b in range(_B):
            store_copy(pc, b).start()
        if pc + 2 < _NCHUNK:
            fill_copy(pc + 2).start()

    for b in range(_B):
        store_copy(_NCHUNK - 1, b).wait()


def kernel(inp, table, is_training):
    del is_training
    pe = jnp.asarray(_PE_HOST)
    out = _emb_kernel(table, inp.reshape(_B * _S), pe)
    return out.reshape(_B, _S, _D)
